# baseline (speedup 1.0000x reference)
import functools

import jax
import jax.numpy as jnp
from jax import lax
from jax.experimental import pallas as pl
from jax.experimental.pallas import tpu as pltpu
from jax.experimental.pallas import tpu_sc as plsc

NOISE_RATIO = 0.1
NOISE_PROB = 0.5
MAX_STEP = 30

_H = 512
_W = 512
_P = _H * _W
_N = int(NOISE_RATIO * _P)
_NPAD = ((_N + 15) // 16) * 16

_NW = 32
_CH = _P // _NW
_LANES = 16
_PLANES = 96
_NBUF = 3

_sc_mesh = plsc.VectorSubcoreMesh(core_axis_name="c", subcore_axis_name="s")


@functools.partial(
    pl.kernel,
    mesh=_sc_mesh,
    out_type=jax.ShapeDtypeStruct((_PLANES, _P), jnp.float32),
    scratch_types=[
        pltpu.VMEM((_NPAD,), jnp.int32),
        pltpu.VMEM((_NPAD,), jnp.float32),
        pltpu.VMEM((_CH,), jnp.float32),
        pltpu.VMEM((_CH,), jnp.float32),
        pltpu.VMEM((_CH,), jnp.float32),
        pltpu.VMEM((_CH,), jnp.float32),
        pltpu.SemaphoreType.DMA,
        pltpu.SemaphoreType.DMA,
        pltpu.SemaphoreType.DMA,
        pltpu.SemaphoreType.DMA,
        pltpu.SemaphoreType.DMA,
        pltpu.SemaphoreType.DMA,
    ],
    compiler_params=pltpu.CompilerParams(needs_layout_passes=False),
)
def _noise_kernel(img_hbm, idx_hbm, vals_hbm, out_hbm,
                  idx_v, vals_v, maskb, b0, b1, b2,
                  si0, si1, si2, so0, so1, so2):
    wid = lax.axis_index("s") * 2 + lax.axis_index("c")
    lo = wid * _CH
    bufs = (b0, b1, b2)
    sis = (si0, si1, si2)
    sos = (so0, so1, so2)

    def in_copy(g, b):
        return pltpu.make_async_copy(
            img_hbm.at[g, pl.ds(lo, _CH)], bufs[b], sis[b])

    def out_copy(g, b):
        return pltpu.make_async_copy(
            bufs[b], out_hbm.at[g, pl.ds(lo, _CH)], sos[b])

    for b in range(_NBUF):
        in_copy(jnp.int32(b), b).start()

    pltpu.sync_copy(idx_hbm, idx_v)
    pltpu.sync_copy(vals_hbm, vals_v)

    ones = jnp.full((_LANES,), 1.0, jnp.float32)

    def init_body(i, carry):
        maskb[pl.ds(i * _LANES, _LANES)] = ones
        return carry

    lax.fori_loop(0, _CH // _LANES, init_body, 0)

    lov = jnp.full((_LANES,), lo, jnp.int32)
    hiv = lov + _CH
    zero = jnp.zeros((_LANES,), jnp.int32)

    def scatter_body(i, carry):
        idx = idx_v[pl.ds(i * _LANES, _LANES)]
        v = vals_v[pl.ds(i * _LANES, _LANES)]
        m = (idx >= lov) & (idx < hiv)
        local = jnp.where(m, idx - lov, zero)
        plsc.store_scatter(maskb, [local], v, mask=m)
        return carry

    lax.fori_loop(0, _NPAD // _LANES, scatter_body, 0)

    def mul_body(buf):
        def body(i, carry):
            x = buf[pl.ds(i * _LANES, _LANES)]
            m = maskb[pl.ds(i * _LANES, _LANES)]
            buf[pl.ds(i * _LANES, _LANES)] = jnp.minimum(
                jnp.maximum(x * m, 0.0), 1.0)
            return carry
        lax.fori_loop(0, _CH // _LANES, body, 0)

    def outer(t, carry):
        for b in range(_NBUF):
            g = t * _NBUF + b
            in_copy(g, b).wait()
            mul_body(bufs[b])
            out_copy(g, b).start()
            g2 = g + 2
            b2 = (b + 2) % _NBUF

            @pl.when((g2 >= _NBUF) & (g2 < _PLANES))
            def _():
                out_copy(g2 - _NBUF, b2).wait()
                in_copy(g2, b2).start()
        return carry

    lax.fori_loop(0, _PLANES // _NBUF, outer, 0)

    for b in range(_NBUF):
        out_copy(jnp.int32(_PLANES - _NBUF + b), b).wait()


def kernel(marked_img, now_step):
    B, C, H, W = marked_img.shape
    num_noisy_pixels = _N

    key = jax.random.key(42)
    kp, kn = jax.random.split(key)
    indices = jax.random.permutation(kp, H * W)[:num_noisy_pixels]
    indices = indices.astype(jnp.int32)
    random_noise = jax.random.uniform(kn, (num_noisy_pixels,), dtype=jnp.float32)
    base_vals = jnp.where(random_noise < NOISE_PROB, 1.0, 0.0).astype(jnp.float32)

    noise_ratio_t = jnp.minimum(now_step / MAX_STEP, 1.0) * NOISE_RATIO
    num_noisy_pixels_t = noise_ratio_t * H * W
    vals = jnp.where(
        jnp.arange(num_noisy_pixels) < num_noisy_pixels_t, base_vals, 1.0
    ).astype(jnp.float32)

    pad = _NPAD - num_noisy_pixels
    idx_full = jnp.concatenate([indices, jnp.full((pad,), _P, jnp.int32)])
    vals_full = jnp.concatenate([vals, jnp.ones((pad,), jnp.float32)])

    img2 = marked_img.reshape(B * C, H * W)
    out2 = _noise_kernel(img2, idx_full, vals_full)
    return out2.reshape(B, C, H, W)


# speedup vs baseline: 1.5863x; 1.5863x over previous
import functools

import jax
import jax.numpy as jnp
from jax import lax
from jax.experimental import pallas as pl
from jax.experimental.pallas import tpu as pltpu
from jax.experimental.pallas import tpu_sc as plsc

NOISE_RATIO = 0.1
NOISE_PROB = 0.5
MAX_STEP = 30

_H = 512
_W = 512
_P = _H * _W
_N = int(NOISE_RATIO * _P)
_NPAD = ((_N + 127) // 128) * 128

_NW = 32
_RW = _H // _NW
_LANES = 16
_B = 32
_C = 3
_NBUF = 3

_sc_mesh = plsc.VectorSubcoreMesh(core_axis_name="c", subcore_axis_name="s")


@functools.partial(
    pl.kernel,
    mesh=_sc_mesh,
    out_type=jax.ShapeDtypeStruct((_B, _C, _H, _W), jnp.float32),
    scratch_types=[
        pltpu.VMEM((_NPAD,), jnp.int32),
        pltpu.VMEM((_NPAD,), jnp.float32),
        pltpu.VMEM((_RW, _W), jnp.float32),
        pltpu.VMEM((_RW, _W), jnp.float32),
        pltpu.VMEM((_RW, _W), jnp.float32),
        pltpu.VMEM((_RW, _W), jnp.float32),
        pltpu.SemaphoreType.DMA,
        pltpu.SemaphoreType.DMA,
        pltpu.SemaphoreType.DMA,
        pltpu.SemaphoreType.DMA,
        pltpu.SemaphoreType.DMA,
        pltpu.SemaphoreType.DMA,
    ],
    compiler_params=pltpu.CompilerParams(
        needs_layout_passes=False, use_tc_tiling_on_sc=True),
)
def _noise_kernel(img_hbm, idx_hbm, vals_hbm, out_hbm,
                  idx_v, vals_v, maskb, b0, b1, b2,
                  si0, si1, si2, so0, so1, so2):
    wid = lax.axis_index("s") * 2 + lax.axis_index("c")
    r0 = wid * _RW
    bufs = (b0, b1, b2)
    sis = (si0, si1, si2)
    sos = (so0, so1, so2)

    def in_copy(t, u, b):
        return pltpu.make_async_copy(
            img_hbm.at[t, u, pl.ds(r0, _RW), :], bufs[b], sis[b])

    def out_copy(t, u, b):
        return pltpu.make_async_copy(
            bufs[b], out_hbm.at[t, u, pl.ds(r0, _RW), :], sos[b])

    for u in range(_NBUF):
        in_copy(jnp.int32(0), u, u).start()

    pltpu.sync_copy(idx_hbm, idx_v)
    pltpu.sync_copy(vals_hbm, vals_v)

    ones = jnp.full((_LANES,), 1.0, jnp.float32)

    for r in range(_RW):
        @plsc.parallel_loop(0, _W // _LANES, unroll=8)
        def _init(i):
            maskb[r, pl.ds(i * _LANES, _LANES)] = ones

    lov = jnp.full((_LANES,), r0 * _W, jnp.int32)
    hiv = lov + _RW * _W
    zero = jnp.zeros((_LANES,), jnp.int32)

    @plsc.parallel_loop(0, _NPAD // _LANES, unroll=4)
    def _scatter(i):
        idx = idx_v[pl.ds(i * _LANES, _LANES)]
        v = vals_v[pl.ds(i * _LANES, _LANES)]
        m = (idx >= lov) & (idx < hiv)
        local = jnp.where(m, idx - lov, zero)
        lr = lax.shift_right_logical(local, 9)
        col = lax.bitwise_and(local, jnp.full((_LANES,), _W - 1, jnp.int32))
        plsc.store_scatter(maskb, [lr, col], v, mask=m)

    def mul_body(buf):
        for r in range(_RW):
            @plsc.parallel_loop(0, _W // _LANES, unroll=8)
            def _mul(i):
                x = buf[r, pl.ds(i * _LANES, _LANES)]
                m = maskb[r, pl.ds(i * _LANES, _LANES)]
                buf[r, pl.ds(i * _LANES, _LANES)] = jnp.minimum(
                    jnp.maximum(x * m, 0.0), 1.0)

    def outer(t, carry):
        in_copy(t, 0, 0).wait()
        mul_body(b0)
        out_copy(t, 0, 0).start()

        @pl.when(t >= 1)
        def _():
            out_copy(t - 1, 2, 2).wait()
            in_copy(t, 2, 2).start()

        in_copy(t, 1, 1).wait()
        mul_body(b1)
        out_copy(t, 1, 1).start()

        @pl.when(t < _B - 1)
        def _():
            out_copy(t, 0, 0).wait()
            in_copy(t + 1, 0, 0).start()

        in_copy(t, 2, 2).wait()
        mul_body(b2)
        out_copy(t, 2, 2).start()

        @pl.when(t < _B - 1)
        def _():
            out_copy(t, 1, 1).wait()
            in_copy(t + 1, 1, 1).start()

        return carry

    lax.fori_loop(0, _B, outer, 0)

    for u in range(_NBUF):
        out_copy(jnp.int32(_B - 1), u, u).wait()


def kernel(marked_img, now_step):
    B, C, H, W = marked_img.shape
    num_noisy_pixels = _N

    key = jax.random.key(42)
    kp, kn = jax.random.split(key)
    indices = jax.random.permutation(kp, H * W)[:num_noisy_pixels]
    indices = indices.astype(jnp.int32)
    random_noise = jax.random.uniform(kn, (num_noisy_pixels,), dtype=jnp.float32)
    base_vals = jnp.where(random_noise < NOISE_PROB, 1.0, 0.0).astype(jnp.float32)

    noise_ratio_t = jnp.minimum(now_step / MAX_STEP, 1.0) * NOISE_RATIO
    num_noisy_pixels_t = noise_ratio_t * H * W
    vals = jnp.where(
        jnp.arange(num_noisy_pixels) < num_noisy_pixels_t, base_vals, 1.0
    ).astype(jnp.float32)

    pad = _NPAD - num_noisy_pixels
    idx_full = jnp.concatenate([indices, jnp.full((pad,), _P, jnp.int32)])
    vals_full = jnp.concatenate([vals, jnp.ones((pad,), jnp.float32)])

    return _noise_kernel(marked_img, idx_full, vals_full)


# speedup vs baseline: 1.7375x; 1.0953x over previous
import functools

import jax
import jax.numpy as jnp
from jax import lax
from jax.experimental import pallas as pl
from jax.experimental.pallas import tpu as pltpu
from jax.experimental.pallas import tpu_sc as plsc

NOISE_RATIO = 0.1
NOISE_PROB = 0.5
MAX_STEP = 30

_H = 512
_W = 512
_P = _H * _W
_N = int(NOISE_RATIO * _P)
_NPAD = ((_N + 127) // 128) * 128

_NW = 32
_RW = _H // _NW
_LANES = 16

_sc_mesh = plsc.VectorSubcoreMesh(core_axis_name="c", subcore_axis_name="s")


@functools.partial(
    pl.kernel,
    mesh=_sc_mesh,
    out_type=jax.ShapeDtypeStruct((_H, _W), jnp.float32),
    scratch_types=[
        pltpu.VMEM((_NPAD,), jnp.int32),
        pltpu.VMEM((_NPAD,), jnp.float32),
        pltpu.VMEM((_RW, _W), jnp.float32),
    ],
    compiler_params=pltpu.CompilerParams(
        needs_layout_passes=False, use_tc_tiling_on_sc=True),
)
def _mask_build(idx_hbm, vals_hbm, out_hbm, idx_v, vals_v, maskb):
    wid = lax.axis_index("s") * 2 + lax.axis_index("c")
    r0 = wid * _RW

    pltpu.sync_copy(idx_hbm, idx_v)
    pltpu.sync_copy(vals_hbm, vals_v)

    ones = jnp.full((_LANES,), 1.0, jnp.float32)

    for r in range(_RW):
        @plsc.parallel_loop(0, _W // _LANES, unroll=8)
        def _init(i):
            maskb[r, pl.ds(i * _LANES, _LANES)] = ones

    lov = jnp.full((_LANES,), r0 * _W, jnp.int32)
    hiv = lov + _RW * _W
    zero = jnp.zeros((_LANES,), jnp.int32)

    @plsc.parallel_loop(0, _NPAD // _LANES, unroll=4)
    def _scatter(i):
        idx = idx_v[pl.ds(i * _LANES, _LANES)]
        v = vals_v[pl.ds(i * _LANES, _LANES)]
        m = (idx >= lov) & (idx < hiv)
        local = jnp.where(m, idx - lov, zero)
        lr = lax.shift_right_logical(local, 9)
        col = lax.bitwise_and(local, jnp.full((_LANES,), _W - 1, jnp.int32))
        plsc.store_scatter(maskb, [lr, col], v, mask=m)

    pltpu.sync_copy(maskb, out_hbm.at[pl.ds(r0, _RW), :])


def _tc_body(img_ref, mask_ref, out_ref):
    out_ref[...] = jnp.clip(img_ref[...] * mask_ref[...][None, :, :], 0.0, 1.0)


def kernel(marked_img, now_step):
    B, C, H, W = marked_img.shape
    num_noisy_pixels = _N

    key = jax.random.key(42)
    kp, kn = jax.random.split(key)
    indices = jax.random.permutation(kp, H * W)[:num_noisy_pixels]
    indices = indices.astype(jnp.int32)
    random_noise = jax.random.uniform(kn, (num_noisy_pixels,), dtype=jnp.float32)
    base_vals = jnp.where(random_noise < NOISE_PROB, 1.0, 0.0).astype(jnp.float32)

    noise_ratio_t = jnp.minimum(now_step / MAX_STEP, 1.0) * NOISE_RATIO
    num_noisy_pixels_t = noise_ratio_t * H * W
    vals = jnp.where(
        jnp.arange(num_noisy_pixels) < num_noisy_pixels_t, base_vals, 1.0
    ).astype(jnp.float32)

    pad = _NPAD - num_noisy_pixels
    idx_full = jnp.concatenate([indices, jnp.full((pad,), _P, jnp.int32)])
    vals_full = jnp.concatenate([vals, jnp.ones((pad,), jnp.float32)])

    mask2d = _mask_build(idx_full, vals_full)

    img3 = marked_img.reshape(B * C, H, W)
    blk = 12
    out3 = pl.pallas_call(
        _tc_body,
        grid=(B * C // blk,),
        in_specs=[
            pl.BlockSpec((blk, H, W), lambda i: (i, 0, 0)),
            pl.BlockSpec((H, W), lambda i: (0, 0)),
        ],
        out_specs=pl.BlockSpec((blk, H, W), lambda i: (i, 0, 0)),
        out_shape=jax.ShapeDtypeStruct((B * C, H, W), jnp.float32),
    )(img3, mask2d)
    return out3.reshape(B, C, H, W)
